# Initial kernel scaffold; baseline (speedup 1.0000x reference)
import functools

import jax
import jax.numpy as jnp
from jax import lax
from jax.experimental import pallas as pl
from jax.experimental.pallas import tpu as pltpu
from jax.experimental.pallas import tpu_sc as plsc

B, C, N, K, O = 2, 128, 4096, 16, 128
TS = 512
NW = 32
RPW = (B * N) // NW
NV = N // 16


def _scores_body(xr_ref, xc_ref, s_ref):
    xr = xr_ref[0]
    xc = xc_ref[0]
    inner = lax.dot_general(xr, xc, (((0,), (0,)), ((), ())),
                            preferred_element_type=jnp.float32)
    sqc = jnp.sum(xc * xc, axis=0, keepdims=True)
    s_ref[0] = 2.0 * inner - sqc


def _scores(x2d):
    return pl.pallas_call(
        _scores_body,
        grid=(B, N // TS, N // TS),
        in_specs=[
            pl.BlockSpec((1, C, TS), lambda b, i, j: (b, 0, i)),
            pl.BlockSpec((1, C, TS), lambda b, i, j: (b, 0, j)),
        ],
        out_specs=pl.BlockSpec((1, TS, TS), lambda b, i, j: (b, i, j)),
        out_shape=jax.ShapeDtypeStruct((B, N, N), jnp.float32),
    )(x2d, x2d)


def _proj_body(xc_ref, wd_ref, wb_ref, wab_ref, be_ref, e1_ref, e2_ref, a2_ref):
    F = xc_ref[0]
    dims = (((0,), (1,)), ((), ()))
    e1_ref[0] = lax.dot_general(F, wd_ref[...], dims,
                                preferred_element_type=jnp.float32) + be_ref[...]
    e2_ref[0] = lax.dot_general(F, wb_ref[...], dims,
                                preferred_element_type=jnp.float32)
    a2_ref[0] = lax.dot_general(F, wab_ref[...], dims,
                                preferred_element_type=jnp.float32)


def _proj(x2d, wd, wb, wab, be):
    return pl.pallas_call(
        _proj_body,
        grid=(B, N // TS),
        in_specs=[
            pl.BlockSpec((1, C, TS), lambda b, i: (b, 0, i)),
            pl.BlockSpec((O, C), lambda b, i: (0, 0)),
            pl.BlockSpec((O, C), lambda b, i: (0, 0)),
            pl.BlockSpec((O, C), lambda b, i: (0, 0)),
            pl.BlockSpec((1, O), lambda b, i: (0, 0)),
        ],
        out_specs=[
            pl.BlockSpec((1, TS, O), lambda b, i: (b, i, 0)),
            pl.BlockSpec((1, TS, O), lambda b, i: (b, i, 0)),
            pl.BlockSpec((1, TS, O), lambda b, i: (b, i, 0)),
        ],
        out_shape=[
            jax.ShapeDtypeStruct((B, N, O), jnp.float32),
            jax.ShapeDtypeStruct((B, N, O), jnp.float32),
            jax.ShapeDtypeStruct((B, N, O), jnp.float32),
        ],
    )(x2d, wd, wb, wab, be)


def _sc_attend_body(s_hbm, e1_hbm, e2_hbm, a2_hbm, out_hbm,
                    srow, idxv, a2b, e2b, e1b, ob, semA, semE):
    wid = lax.axis_index("s") * 2 + lax.axis_index("c")
    base = wid * RPW
    lanes = lax.broadcasted_iota(jnp.int32, (K,), 0)

    def row_body(r, _):
        g = base + r
        b = g // N
        pltpu.sync_copy(s_hbm.at[g], srow)

        def vb(i, carry):
            T, Ti = carry
            c = srow[pl.ds(i * 16, 16)]
            idx = lanes + i * 16
            sc, sci = plsc.sort_key_val(c, idx, descending=True)
            m = T >= sc
            nT = jnp.where(m, T, sc)
            nTi = jnp.where(m, Ti, sci)
            return plsc.sort_key_val(nT, nTi)

        T0 = jnp.full((K,), -3e38, jnp.float32)
        Ti0 = jnp.zeros((K,), jnp.int32)
        _, Ti = lax.fori_loop(0, NV, vb, (T0, Ti0))
        idxv[...] = Ti + b * N
        cpA = pltpu.async_copy(a2_hbm.at[idxv], a2b, semA)
        cpE = pltpu.async_copy(e2_hbm.at[idxv], e2b, semE)
        pltpu.sync_copy(e1_hbm.at[g], e1b)
        cpA.wait()
        cpE.wait()
        for gg in range(O // 16):
            sl = pl.ds(gg * 16, 16)
            avs = [a2b[j, sl] for j in range(K)]
            mx = avs[0]
            for j in range(1, K):
                mx = jnp.maximum(mx, avs[j])
            es = [jnp.exp(a - mx) for a in avs]
            ssum = es[0]
            for j in range(1, K):
                ssum = ssum + es[j]
            acc = es[0] * e2b[0, sl]
            for j in range(1, K):
                acc = acc + es[j] * e2b[j, sl]
            ob[sl] = acc / ssum + e1b[sl]
        pltpu.sync_copy(ob, out_hbm.at[g])
        return 0

    lax.fori_loop(0, RPW, row_body, 0)


_sc_attend = functools.partial(
    pl.kernel,
    mesh=plsc.VectorSubcoreMesh(core_axis_name="c", subcore_axis_name="s"),
    out_type=jax.ShapeDtypeStruct((B * N, O), jnp.float32),
    scratch_types=[
        pltpu.VMEM((N,), jnp.float32),
        pltpu.VMEM((K,), jnp.int32),
        pltpu.VMEM((K, O), jnp.float32),
        pltpu.VMEM((K, O), jnp.float32),
        pltpu.VMEM((O,), jnp.float32),
        pltpu.VMEM((O,), jnp.float32),
        pltpu.SemaphoreType.DMA,
        pltpu.SemaphoreType.DMA,
    ],
)(_sc_attend_body)


def _tr_body(i_ref, o_ref):
    o_ref[0] = i_ref[0].T


def _transpose(out_t):
    return pl.pallas_call(
        _tr_body,
        grid=(B, N // TS),
        in_specs=[pl.BlockSpec((1, TS, O), lambda b, i: (b, i, 0))],
        out_specs=pl.BlockSpec((1, O, TS), lambda b, i: (b, 0, i)),
        out_shape=jax.ShapeDtypeStruct((B, O, N), jnp.float32),
    )(out_t)


def kernel(x, W_edge, b_edge, W_att, b_att):
    x2d = x[..., 0]
    S = _scores(x2d)
    Wb = W_edge[:, C:]
    E1, E2, A2 = _proj(x2d, W_edge[:, :C] - Wb, Wb, W_att[:, C:],
                       b_edge.reshape(1, O))
    out_t = _sc_attend(S.reshape(B * N, N), E1.reshape(B * N, O),
                       E2.reshape(B * N, O), A2.reshape(B * N, O))
    out = _transpose(out_t.reshape(B, N, O))
    return out[..., None]


# speedup vs baseline: 7.2793x; 7.2793x over previous
import functools

import jax
import jax.numpy as jnp
from jax import lax
from jax.experimental import pallas as pl
from jax.experimental.pallas import tpu as pltpu
from jax.experimental.pallas import tpu_sc as plsc

B, C, N, K, O = 2, 128, 4096, 16, 128
TS = 512
NW = 32
RPW = (B * N) // NW
NV = N // 16


def _scores_body(xr_ref, xc_ref, s_ref):
    xr = xr_ref[0]
    xc = xc_ref[0]
    inner = lax.dot_general(xr, xc, (((0,), (0,)), ((), ())),
                            preferred_element_type=jnp.float32)
    sqc = jnp.sum(xc * xc, axis=0, keepdims=True)
    s_ref[0] = 2.0 * inner - sqc


def _scores(x2d):
    return pl.pallas_call(
        _scores_body,
        grid=(B, N // TS, N // TS),
        in_specs=[
            pl.BlockSpec((1, C, TS), lambda b, i, j: (b, 0, i)),
            pl.BlockSpec((1, C, TS), lambda b, i, j: (b, 0, j)),
        ],
        out_specs=pl.BlockSpec((1, TS, TS), lambda b, i, j: (b, i, j)),
        out_shape=jax.ShapeDtypeStruct((B, N, N), jnp.float32),
    )(x2d, x2d)


def _proj_body(xc_ref, wd_ref, wb_ref, wab_ref, be_ref, e1_ref, e2_ref, a2_ref):
    F = xc_ref[0]
    dims = (((0,), (1,)), ((), ()))
    e1_ref[0] = lax.dot_general(F, wd_ref[...], dims,
                                preferred_element_type=jnp.float32) + be_ref[...]
    e2_ref[0] = lax.dot_general(F, wb_ref[...], dims,
                                preferred_element_type=jnp.float32)
    a2_ref[0] = lax.dot_general(F, wab_ref[...], dims,
                                preferred_element_type=jnp.float32)


def _proj(x2d, wd, wb, wab, be):
    return pl.pallas_call(
        _proj_body,
        grid=(B, N // TS),
        in_specs=[
            pl.BlockSpec((1, C, TS), lambda b, i: (b, 0, i)),
            pl.BlockSpec((O, C), lambda b, i: (0, 0)),
            pl.BlockSpec((O, C), lambda b, i: (0, 0)),
            pl.BlockSpec((O, C), lambda b, i: (0, 0)),
            pl.BlockSpec((1, O), lambda b, i: (0, 0)),
        ],
        out_specs=[
            pl.BlockSpec((1, TS, O), lambda b, i: (b, i, 0)),
            pl.BlockSpec((1, TS, O), lambda b, i: (b, i, 0)),
            pl.BlockSpec((1, TS, O), lambda b, i: (b, i, 0)),
        ],
        out_shape=[
            jax.ShapeDtypeStruct((B, N, O), jnp.float32),
            jax.ShapeDtypeStruct((B, N, O), jnp.float32),
            jax.ShapeDtypeStruct((B, N, O), jnp.float32),
        ],
    )(x2d, wd, wb, wab, be)


def _sc_attend_body(s_hbm, e1_hbm, e2_hbm, a2_hbm, out_hbm,
                    srow, idxv, a2b, e2b, e1b, ob, semA, semE):
    wid = lax.axis_index("s") * 2 + lax.axis_index("c")
    base = wid * RPW
    lanes = lax.broadcasted_iota(jnp.int32, (K,), 0)

    def row_body(r, _):
        g = base + r
        b = g // N
        pltpu.sync_copy(s_hbm.at[g], srow)

        def vb(i, carry):
            T, Ti = carry
            c = srow[pl.ds(i * 16, 16)]
            idx = lanes + i * 16
            sc, sci = plsc.sort_key_val(c, idx, descending=True)
            m = T >= sc
            nT = jnp.where(m, T, sc)
            nTi = jnp.where(m, Ti, sci)
            rT, rTi = plsc.sort_key_val(nT, nTi)
            return (rT, rTi)

        T0 = jnp.full((K,), -3e38, jnp.float32)
        Ti0 = jnp.zeros((K,), jnp.int32)
        _, Ti = lax.fori_loop(0, NV, vb, (T0, Ti0))
        idxv[...] = Ti + b * N
        cpA = pltpu.async_copy(a2_hbm.at[idxv], a2b, semA)
        cpE = pltpu.async_copy(e2_hbm.at[idxv], e2b, semE)
        pltpu.sync_copy(e1_hbm.at[g], e1b)
        cpA.wait()
        cpE.wait()
        for gg in range(O // 16):
            sl = pl.ds(gg * 16, 16)
            avs = [a2b[j, sl] for j in range(K)]
            mx = avs[0]
            for j in range(1, K):
                mx = jnp.maximum(mx, avs[j])
            es = [jnp.exp(a - mx) for a in avs]
            ssum = es[0]
            for j in range(1, K):
                ssum = ssum + es[j]
            acc = es[0] * e2b[0, sl]
            for j in range(1, K):
                acc = acc + es[j] * e2b[j, sl]
            ob[sl] = acc / ssum + e1b[sl]
        pltpu.sync_copy(ob, out_hbm.at[g])
        return 0

    lax.fori_loop(0, RPW, row_body, 0)


_sc_attend = functools.partial(
    pl.kernel,
    mesh=plsc.VectorSubcoreMesh(core_axis_name="c", subcore_axis_name="s"),
    compiler_params=pltpu.CompilerParams(needs_layout_passes=False),
    out_type=jax.ShapeDtypeStruct((B * N, O), jnp.float32),
    scratch_types=[
        pltpu.VMEM((N,), jnp.float32),
        pltpu.VMEM((K,), jnp.int32),
        pltpu.VMEM((K, O), jnp.float32),
        pltpu.VMEM((K, O), jnp.float32),
        pltpu.VMEM((O,), jnp.float32),
        pltpu.VMEM((O,), jnp.float32),
        pltpu.SemaphoreType.DMA,
        pltpu.SemaphoreType.DMA,
    ],
)(_sc_attend_body)


def _tr_body(i_ref, o_ref):
    o_ref[0] = i_ref[0].T


def _transpose(out_t):
    return pl.pallas_call(
        _tr_body,
        grid=(B, N // TS),
        in_specs=[pl.BlockSpec((1, TS, O), lambda b, i: (b, i, 0))],
        out_specs=pl.BlockSpec((1, O, TS), lambda b, i: (b, 0, i)),
        out_shape=jax.ShapeDtypeStruct((B, O, N), jnp.float32),
    )(out_t)


def kernel(x, W_edge, b_edge, W_att, b_att):
    x2d = x[..., 0]
    S = _scores(x2d)
    Wb = W_edge[:, C:]
    E1, E2, A2 = _proj(x2d, W_edge[:, :C] - Wb, Wb, W_att[:, C:],
                       b_edge.reshape(1, O))
    out_t = _sc_attend(S.reshape(B * N, N), E1.reshape(B * N, O),
                       E2.reshape(B * N, O), A2.reshape(B * N, O))
    out = _transpose(out_t.reshape(B, N, O))
    return out[..., None]


# speedup vs baseline: 23.3263x; 3.2044x over previous
import functools

import jax
import jax.numpy as jnp
from jax import lax
from jax.experimental import pallas as pl
from jax.experimental.pallas import tpu as pltpu
from jax.experimental.pallas import tpu_sc as plsc

B, C, N, K, O = 2, 128, 4096, 16, 128
TS = 512
NW = 32
RPW = (B * N) // NW
NV = N // 16


def _scores_body(xr_ref, xc_ref, s_ref):
    xr = xr_ref[0]
    xc = xc_ref[0]
    inner = lax.dot_general(xr, xc, (((0,), (0,)), ((), ())),
                            preferred_element_type=jnp.float32)
    sqc = jnp.sum(xc * xc, axis=0, keepdims=True)
    s_ref[0] = 2.0 * inner - sqc


def _scores(x2d):
    return pl.pallas_call(
        _scores_body,
        grid=(B, N // TS, N // TS),
        in_specs=[
            pl.BlockSpec((1, C, TS), lambda b, i, j: (b, 0, i)),
            pl.BlockSpec((1, C, TS), lambda b, i, j: (b, 0, j)),
        ],
        out_specs=pl.BlockSpec((1, TS, TS), lambda b, i, j: (b, i, j)),
        out_shape=jax.ShapeDtypeStruct((B, N, N), jnp.float32),
    )(x2d, x2d)


def _proj_body(xc_ref, wd_ref, wb_ref, wab_ref, be_ref, e1_ref, ae_ref):
    F = xc_ref[0]
    dims = (((0,), (1,)), ((), ()))
    e1_ref[0] = lax.dot_general(F, wd_ref[...], dims,
                                preferred_element_type=jnp.float32) + be_ref[...]
    ae_ref[0, :, :O] = lax.dot_general(F, wab_ref[...], dims,
                                       preferred_element_type=jnp.float32)
    ae_ref[0, :, O:] = lax.dot_general(F, wb_ref[...], dims,
                                       preferred_element_type=jnp.float32)


def _proj(x2d, wd, wb, wab, be):
    return pl.pallas_call(
        _proj_body,
        grid=(B, N // TS),
        in_specs=[
            pl.BlockSpec((1, C, TS), lambda b, i: (b, 0, i)),
            pl.BlockSpec((O, C), lambda b, i: (0, 0)),
            pl.BlockSpec((O, C), lambda b, i: (0, 0)),
            pl.BlockSpec((O, C), lambda b, i: (0, 0)),
            pl.BlockSpec((1, O), lambda b, i: (0, 0)),
        ],
        out_specs=[
            pl.BlockSpec((1, TS, O), lambda b, i: (b, i, 0)),
            pl.BlockSpec((1, TS, 2 * O), lambda b, i: (b, i, 0)),
        ],
        out_shape=[
            jax.ShapeDtypeStruct((B, N, O), jnp.float32),
            jax.ShapeDtypeStruct((B, N, 2 * O), jnp.float32),
        ],
    )(x2d, wd, wb, wab, be)


G = 8
NGR = RPW // G


def _sc_attend_body(s_hbm, e1_hbm, ae_hbm, out_hbm,
                    sbuf, idxv, gbuf, e1b, ob, semS, semG, semE):
    wid = lax.axis_index("s") * 2 + lax.axis_index("c")
    rbase = wid * RPW
    bofs = (rbase // N) * N
    lanes = lax.broadcasted_iota(jnp.int32, (16,), 0)

    def scores_dma(g, p):
        return pltpu.make_async_copy(
            s_hbm.at[pl.ds(rbase + g * G, G)], sbuf.at[p], semS.at[p])

    def gather_dma(p):
        return pltpu.make_async_copy(ae_hbm.at[idxv.at[p]], gbuf, semG)

    def e1_dma(g):
        return pltpu.make_async_copy(
            e1_hbm.at[pl.ds(rbase + g * G, G)], e1b, semE)

    def topk_group(p):
        nT0 = jnp.full((16,), -3e38, jnp.float32)
        nTi0 = jnp.zeros((16,), jnp.int32)
        init = tuple([nT0] * G + [nTi0] * G)

        def vb(i, carry):
            Ts = list(carry[:G])
            Tis = list(carry[G:])
            idx0 = lanes + i * 16
            for r in range(G):
                c = sbuf[p, r, pl.ds(i * 16, 16)]
                sc, sci = plsc.sort_key_val(c, idx0, descending=True)
                m = Ts[r] >= sc
                nT = jnp.where(m, Ts[r], sc)
                nTi = jnp.where(m, Tis[r], sci)
                Ts[r], Tis[r] = plsc.sort_key_val(nT, nTi)
            return tuple(Ts) + tuple(Tis)

        carry = lax.fori_loop(0, NV, vb, init)
        for r in range(G):
            idxv[p, pl.ds(r * K, K)] = carry[G + r] + bofs

    def softmax_group(g):
        def row(r, _):
            rb = r * K
            for q in range(O // 16):
                sa = pl.ds(q * 16, 16)
                se = pl.ds(O + q * 16, 16)
                avs = [gbuf[rb + j, sa] for j in range(K)]
                mx = avs[0]
                for j in range(1, K):
                    mx = jnp.maximum(mx, avs[j])
                es = [jnp.exp(a - mx) for a in avs]
                ssum = es[0]
                for j in range(1, K):
                    ssum = ssum + es[j]
                acc = es[0] * gbuf[rb, se]
                for j in range(1, K):
                    acc = acc + es[j] * gbuf[rb + j, se]
                ob[r, sa] = acc / ssum + e1b[r, sa]
            return 0

        lax.fori_loop(0, G, row, 0)
        pltpu.sync_copy(ob, out_hbm.at[pl.ds(rbase + g * G, G)])

    pltpu.async_copy(s_hbm.at[pl.ds(rbase, G)], sbuf.at[0], semS.at[0])

    def step(g, _):
        p = g % 2

        @pl.when(g < NGR)
        def _():
            scores_dma(g, p).wait()

            @pl.when(g + 1 < NGR)
            def _():
                pltpu.async_copy(s_hbm.at[pl.ds(rbase + (g + 1) * G, G)],
                                 sbuf.at[1 - p], semS.at[1 - p])

            topk_group(p)

        @pl.when(g > 0)
        def _():
            gather_dma(1 - p).wait()
            e1_dma(g - 1).wait()
            softmax_group(g - 1)

        @pl.when(g < NGR)
        def _():
            pltpu.async_copy(ae_hbm.at[idxv.at[p]], gbuf, semG)
            pltpu.async_copy(e1_hbm.at[pl.ds(rbase + g * G, G)], e1b, semE)

        return 0

    lax.fori_loop(0, NGR + 1, step, 0)


_sc_attend = functools.partial(
    pl.kernel,
    mesh=plsc.VectorSubcoreMesh(core_axis_name="c", subcore_axis_name="s"),
    compiler_params=pltpu.CompilerParams(needs_layout_passes=False),
    out_type=jax.ShapeDtypeStruct((B * N, O), jnp.float32),
    scratch_types=[
        pltpu.VMEM((2, G, N), jnp.float32),
        pltpu.VMEM((2, G * K), jnp.int32),
        pltpu.VMEM((G * K, 2 * O), jnp.float32),
        pltpu.VMEM((G, O), jnp.float32),
        pltpu.VMEM((G, O), jnp.float32),
        pltpu.SemaphoreType.DMA((2,)),
        pltpu.SemaphoreType.DMA,
        pltpu.SemaphoreType.DMA,
    ],
)(_sc_attend_body)


def _tr_body(i_ref, o_ref):
    o_ref[0] = i_ref[0].T


def _transpose(out_t):
    return pl.pallas_call(
        _tr_body,
        grid=(B, N // TS),
        in_specs=[pl.BlockSpec((1, TS, O), lambda b, i: (b, i, 0))],
        out_specs=pl.BlockSpec((1, O, TS), lambda b, i: (b, 0, i)),
        out_shape=jax.ShapeDtypeStruct((B, O, N), jnp.float32),
    )(out_t)


def kernel(x, W_edge, b_edge, W_att, b_att):
    x2d = x[..., 0]
    S = _scores(x2d)
    Wb = W_edge[:, C:]
    E1, AE = _proj(x2d, W_edge[:, :C] - Wb, Wb, W_att[:, C:],
                   b_edge.reshape(1, O))
    out_t = _sc_attend(S.reshape(B * N, N), E1.reshape(B * N, O),
                       AE.reshape(B * N, 2 * O))
    out = _transpose(out_t.reshape(B, N, O))
    return out[..., None]


# speedup vs baseline: 25.2642x; 1.0831x over previous
import functools

import jax
import jax.numpy as jnp
from jax import lax
from jax.experimental import pallas as pl
from jax.experimental.pallas import tpu as pltpu
from jax.experimental.pallas import tpu_sc as plsc

B, C, N, K, O = 2, 128, 4096, 16, 128
TS = 512
NW = 32
RPW = (B * N) // NW
NV = N // 16


def _scores_body(xr_ref, xc_ref, s_ref):
    xr = xr_ref[0]
    xc = xc_ref[0]
    inner = lax.dot_general(xr, xc, (((0,), (0,)), ((), ())),
                            preferred_element_type=jnp.float32)
    sqc = jnp.sum(xc * xc, axis=0, keepdims=True)
    s_ref[0] = 2.0 * inner - sqc


def _scores_half(x2d, h):
    return pl.pallas_call(
        _scores_body,
        grid=(1, N // TS, N // TS),
        in_specs=[
            pl.BlockSpec((1, C, TS), lambda b, i, j: (h, 0, i)),
            pl.BlockSpec((1, C, TS), lambda b, i, j: (h, 0, j)),
        ],
        out_specs=pl.BlockSpec((1, TS, TS), lambda b, i, j: (b, i, j)),
        out_shape=jax.ShapeDtypeStruct((1, N, N), jnp.float32),
    )(x2d, x2d)


def _proj_body(xc_ref, wd_ref, wb_ref, wab_ref, be_ref, e1_ref, ae_ref):
    F = xc_ref[0]
    dims = (((0,), (1,)), ((), ()))
    e1_ref[0] = lax.dot_general(F, wd_ref[...], dims,
                                preferred_element_type=jnp.float32) + be_ref[...]
    ae_ref[0, :, :O] = lax.dot_general(F, wab_ref[...], dims,
                                       preferred_element_type=jnp.float32)
    ae_ref[0, :, O:] = lax.dot_general(F, wb_ref[...], dims,
                                       preferred_element_type=jnp.float32)


def _proj(x2d, wd, wb, wab, be):
    return pl.pallas_call(
        _proj_body,
        grid=(B, N // TS),
        in_specs=[
            pl.BlockSpec((1, C, TS), lambda b, i: (b, 0, i)),
            pl.BlockSpec((O, C), lambda b, i: (0, 0)),
            pl.BlockSpec((O, C), lambda b, i: (0, 0)),
            pl.BlockSpec((O, C), lambda b, i: (0, 0)),
            pl.BlockSpec((1, O), lambda b, i: (0, 0)),
        ],
        out_specs=[
            pl.BlockSpec((1, TS, O), lambda b, i: (b, i, 0)),
            pl.BlockSpec((1, TS, 2 * O), lambda b, i: (b, i, 0)),
        ],
        out_shape=[
            jax.ShapeDtypeStruct((B, N, O), jnp.float32),
            jax.ShapeDtypeStruct((B, N, 2 * O), jnp.float32),
        ],
    )(x2d, wd, wb, wab, be)


G = 8


def _make_sc_attend(row0):
    RPWh = N // NW
    NGRh = RPWh // G

    def body(s_hbm, e1_hbm, ae_hbm, out_hbm,
             sbuf, idxv, gbuf, e1b, ob, semS, semG, semE):
        wid = lax.axis_index("s") * 2 + lax.axis_index("c")
        rbase = wid * RPWh
        lanes = lax.broadcasted_iota(jnp.int32, (16,), 0)

        def topk_group(p):
            nT0 = jnp.full((16,), -3e38, jnp.float32)
            nTi0 = jnp.zeros((16,), jnp.int32)
            init = tuple([nT0] * G + [nTi0] * G)

            def vb(i, carry):
                Ts = list(carry[:G])
                Tis = list(carry[G:])
                idx0 = lanes + i * 16
                for r in range(G):
                    c = sbuf[p, r, pl.ds(i * 16, 16)]
                    sc, sci = plsc.sort_key_val(c, idx0, descending=True)
                    m = Ts[r] >= sc
                    nT = jnp.where(m, Ts[r], sc)
                    nTi = jnp.where(m, Tis[r], sci)
                    Ts[r], Tis[r] = plsc.sort_key_val(nT, nTi)
                return tuple(Ts) + tuple(Tis)

            carry = lax.fori_loop(0, NV, vb, init)
            for r in range(G):
                idxv[p, pl.ds(r * K, K)] = carry[G + r] + row0

        def softmax_group(g):
            def row(r, _):
                rb = r * K
                for q in range(O // 16):
                    sa = pl.ds(q * 16, 16)
                    se = pl.ds(O + q * 16, 16)
                    avs = [gbuf[rb + j, sa] for j in range(K)]
                    mx = avs[0]
                    for j in range(1, K):
                        mx = jnp.maximum(mx, avs[j])
                    es = [jnp.exp(a - mx) for a in avs]
                    ssum = es[0]
                    for j in range(1, K):
                        ssum = ssum + es[j]
                    acc = es[0] * gbuf[rb, se]
                    for j in range(1, K):
                        acc = acc + es[j] * gbuf[rb + j, se]
                    ob[r, sa] = acc / ssum + e1b[r, sa]
                return 0

            lax.fori_loop(0, G, row, 0)
            pltpu.sync_copy(ob, out_hbm.at[pl.ds(rbase + g * G, G)])

        pltpu.async_copy(s_hbm.at[pl.ds(rbase, G)], sbuf.at[0], semS.at[0])

        def step(g, _):
            p = g % 2

            @pl.when(g < NGRh)
            def _():
                pltpu.make_async_copy(
                    s_hbm.at[pl.ds(rbase + g * G, G)], sbuf.at[p],
                    semS.at[p]).wait()

                @pl.when(g + 1 < NGRh)
                def _():
                    pltpu.async_copy(
                        s_hbm.at[pl.ds(rbase + (g + 1) * G, G)],
                        sbuf.at[1 - p], semS.at[1 - p])

                topk_group(p)

            @pl.when(g > 0)
            def _():
                pltpu.make_async_copy(
                    ae_hbm.at[idxv.at[1 - p]], gbuf, semG).wait()
                pltpu.make_async_copy(
                    e1_hbm.at[pl.ds(row0 + rbase + (g - 1) * G, G)], e1b,
                    semE).wait()
                softmax_group(g - 1)

            @pl.when(g < NGRh)
            def _():
                pltpu.async_copy(ae_hbm.at[idxv.at[p]], gbuf, semG)
                pltpu.async_copy(
                    e1_hbm.at[pl.ds(row0 + rbase + g * G, G)], e1b, semE)

            return 0

        lax.fori_loop(0, NGRh + 1, step, 0)

    return pl.kernel(
        body,
        mesh=plsc.VectorSubcoreMesh(core_axis_name="c", subcore_axis_name="s"),
        compiler_params=pltpu.CompilerParams(needs_layout_passes=False),
        out_type=jax.ShapeDtypeStruct((N, O), jnp.float32),
        scratch_types=[
            pltpu.VMEM((2, G, N), jnp.float32),
            pltpu.VMEM((2, G * K), jnp.int32),
            pltpu.VMEM((G * K, 2 * O), jnp.float32),
            pltpu.VMEM((G, O), jnp.float32),
            pltpu.VMEM((G, O), jnp.float32),
            pltpu.SemaphoreType.DMA((2,)),
            pltpu.SemaphoreType.DMA,
            pltpu.SemaphoreType.DMA,
        ],
    )


_sc_attend_h = tuple(_make_sc_attend(h * N) for h in range(B))


def _tr_body(i_ref, o_ref):
    o_ref[0] = i_ref[0].T


def _transpose(out_t):
    return pl.pallas_call(
        _tr_body,
        grid=(1, N // TS),
        in_specs=[pl.BlockSpec((1, TS, O), lambda b, i: (b, i, 0))],
        out_specs=pl.BlockSpec((1, O, TS), lambda b, i: (b, 0, i)),
        out_shape=jax.ShapeDtypeStruct((1, O, N), jnp.float32),
    )(out_t)


def kernel(x, W_edge, b_edge, W_att, b_att):
    x2d = x[..., 0]
    Wb = W_edge[:, C:]
    E1, AE = _proj(x2d, W_edge[:, :C] - Wb, Wb, W_att[:, C:],
                   b_edge.reshape(1, O))
    e1f = E1.reshape(B * N, O)
    aef = AE.reshape(B * N, 2 * O)
    halves = []
    for h in range(B):
        S_h = _scores_half(x2d, h)
        o_h = _sc_attend_h[h](S_h.reshape(N, N), e1f, aef)
        halves.append(_transpose(o_h.reshape(1, N, O)))
    out = jnp.concatenate(halves, axis=0)
    return out[..., None]
